# Initial kernel scaffold; baseline (speedup 1.0000x reference)
import functools

import jax
import jax.numpy as jnp
from jax import lax
from jax.experimental import pallas as pl
from jax.experimental.pallas import tpu as pltpu
from jax.experimental.pallas import tpu_sc as plsc

N_NODES = 10000
D_FEAT = 128

NC = 2
NS = 16
NW = NC * NS
CH = 128
LANES = 16

N_PAD = ((N_NODES + NS * CH - 1) // (NS * CH)) * NS * CH
RPS = N_PAD // NS


def _hist_body(src_hbm, tgt_hbm, hist_hbm, src_v, tgt_v, ones_v, zbuf_v,
               hin_sh, hout_sh, *, k_chunks):
    c = lax.axis_index("c")
    s = lax.axis_index("s")
    wid = c * NS + s

    @pl.loop(0, RPS // LANES)
    def _zero(i):
        zbuf_v[pl.ds(i * LANES, LANES)] = jnp.zeros((LANES,), jnp.float32)

    @pl.loop(0, CH // LANES)
    def _ones(i):
        ones_v[pl.ds(i * LANES, LANES)] = jnp.ones((LANES,), jnp.float32)

    pltpu.sync_copy(zbuf_v, hin_sh.at[pl.ds(s * RPS, RPS)])
    pltpu.sync_copy(zbuf_v, hout_sh.at[pl.ds(s * RPS, RPS)])
    plsc.subcore_barrier()

    pltpu.sync_copy(src_hbm.at[wid], src_v)
    pltpu.sync_copy(tgt_hbm.at[wid], tgt_v)

    @pl.loop(0, k_chunks)
    def _scatter(k):
        pltpu.sync_copy(ones_v, hin_sh.at[tgt_v.at[k]], add=True)
        pltpu.sync_copy(ones_v, hout_sh.at[src_v.at[k]], add=True)

    plsc.subcore_barrier()

    pltpu.sync_copy(hin_sh.at[pl.ds(s * RPS, RPS)], zbuf_v)
    pltpu.sync_copy(zbuf_v, hist_hbm.at[c, 0, pl.ds(s * RPS, RPS)])
    pltpu.sync_copy(hout_sh.at[pl.ds(s * RPS, RPS)], zbuf_v)
    pltpu.sync_copy(zbuf_v, hist_hbm.at[c, 1, pl.ds(s * RPS, RPS)])


def _agg_body(g_hbm, src_hbm, tgt_hbm, p_hbm, src_v, tgt_v, buf_a, buf_b,
              acc_sh, sem_a, sem_b, *, k_chunks):
    c = lax.axis_index("c")
    s = lax.axis_index("s")
    wid = c * NS + s

    @pl.loop(0, CH)
    def _zr(i):
        @pl.loop(0, D_FEAT // LANES)
        def _zc(j):
            buf_a[i, pl.ds(j * LANES, LANES)] = jnp.zeros((LANES,), jnp.float32)

    for r in range(RPS // CH):
        pltpu.sync_copy(buf_a, acc_sh.at[pl.ds(s * RPS + r * CH, CH)])
    plsc.subcore_barrier()

    pltpu.sync_copy(src_hbm.at[wid], src_v)
    pltpu.sync_copy(tgt_hbm.at[wid], tgt_v)

    pltpu.async_copy(g_hbm.at[src_v.at[0]], buf_a, sem_a)

    @pl.loop(0, k_chunks - 1)
    def _pipe(k):
        even = (k % 2) == 0

        @pl.when(even)
        def _():
            pltpu.async_copy(g_hbm.at[src_v.at[k + 1]], buf_b, sem_b)
            pltpu.make_async_copy(g_hbm.at[src_v.at[k]], buf_a, sem_a).wait()
            pltpu.sync_copy(buf_a, acc_sh.at[tgt_v.at[k]], add=True)

        @pl.when(jnp.logical_not(even))
        def _():
            pltpu.async_copy(g_hbm.at[src_v.at[k + 1]], buf_a, sem_a)
            pltpu.make_async_copy(g_hbm.at[src_v.at[k]], buf_b, sem_b).wait()
            pltpu.sync_copy(buf_b, acc_sh.at[tgt_v.at[k]], add=True)

    last = k_chunks - 1
    if last % 2 == 0:
        pltpu.make_async_copy(g_hbm.at[src_v.at[last]], buf_a, sem_a).wait()
        pltpu.sync_copy(buf_a, acc_sh.at[tgt_v.at[last]], add=True)
    else:
        pltpu.make_async_copy(g_hbm.at[src_v.at[last]], buf_b, sem_b).wait()
        pltpu.sync_copy(buf_b, acc_sh.at[tgt_v.at[last]], add=True)

    plsc.subcore_barrier()

    for r in range(RPS // CH):
        rows = pl.ds(s * RPS + r * CH, CH)
        pltpu.sync_copy(acc_sh.at[rows], buf_a)
        pltpu.sync_copy(buf_a, p_hbm.at[c, rows])


def _scale_body(hout_ref, h_ref, g_ref):
    deg = hout_ref[0, :] + hout_ref[1, :]
    f = jnp.where(deg > 0.0, lax.rsqrt(deg), 0.0)
    g_ref[...] = h_ref[...] * f[:, None]


def _combine_body(hin_ref, p_ref, o_ref):
    deg = hin_ref[0, :] + hin_ref[1, :]
    f = jnp.where(deg > 0.0, lax.rsqrt(deg), 0.0)
    o_ref[...] = (p_ref[0] + p_ref[1]) * f[:, None]


def kernel(node_embeddings, adjacency_list):
    n_edges = adjacency_list.shape[0]
    k_chunks = -(-n_edges // (NW * CH))
    e_pad = NW * k_chunks * CH

    src = adjacency_list[:, 0].astype(jnp.int32)
    tgt = adjacency_list[:, 1].astype(jnp.int32)
    pad = jnp.full((e_pad - n_edges,), N_NODES, jnp.int32)
    src3 = jnp.concatenate([src, pad]).reshape(NW, k_chunks, CH)
    tgt3 = jnp.concatenate([tgt, pad]).reshape(NW, k_chunks, CH)
    h_pad = jnp.pad(node_embeddings, ((0, N_PAD - N_NODES), (0, 0)))

    mesh = plsc.VectorSubcoreMesh(core_axis_name="c", subcore_axis_name="s")

    hist = pl.kernel(
        functools.partial(_hist_body, k_chunks=k_chunks),
        out_type=jax.ShapeDtypeStruct((NC, 2, N_PAD), jnp.float32),
        mesh=mesh,
        scratch_types=[
            pltpu.VMEM((k_chunks, CH), jnp.int32),
            pltpu.VMEM((k_chunks, CH), jnp.int32),
            pltpu.VMEM((CH,), jnp.float32),
            pltpu.VMEM((RPS,), jnp.float32),
            pltpu.VMEM_SHARED((N_PAD,), jnp.float32),
            pltpu.VMEM_SHARED((N_PAD,), jnp.float32),
        ],
    )(src3, tgt3)

    grid8 = N_PAD // 1280
    g = pl.pallas_call(
        _scale_body,
        grid=(grid8,),
        in_specs=[
            pl.BlockSpec((NC, 1280), lambda i: (0, i)),
            pl.BlockSpec((1280, D_FEAT), lambda i: (i, 0)),
        ],
        out_specs=pl.BlockSpec((1280, D_FEAT), lambda i: (i, 0)),
        out_shape=jax.ShapeDtypeStruct((N_PAD, D_FEAT), jnp.float32),
    )(hist[:, 1, :], h_pad)

    partials = pl.kernel(
        functools.partial(_agg_body, k_chunks=k_chunks),
        out_type=jax.ShapeDtypeStruct((NC, N_PAD, D_FEAT), jnp.float32),
        mesh=mesh,
        scratch_types=[
            pltpu.VMEM((k_chunks, CH), jnp.int32),
            pltpu.VMEM((k_chunks, CH), jnp.int32),
            pltpu.VMEM((CH, D_FEAT), jnp.float32),
            pltpu.VMEM((CH, D_FEAT), jnp.float32),
            pltpu.VMEM_SHARED((N_PAD, D_FEAT), jnp.float32),
            pltpu.SemaphoreType.DMA,
            pltpu.SemaphoreType.DMA,
        ],
    )(g, src3, tgt3)

    out = pl.pallas_call(
        _combine_body,
        grid=(grid8,),
        in_specs=[
            pl.BlockSpec((NC, 1280), lambda i: (0, i)),
            pl.BlockSpec((NC, 1280, D_FEAT), lambda i: (0, i, 0)),
        ],
        out_specs=pl.BlockSpec((1280, D_FEAT), lambda i: (i, 0)),
        out_shape=jax.ShapeDtypeStruct((N_PAD, D_FEAT), jnp.float32),
    )(hist[:, 0, :], partials)

    return out[:N_NODES]


# speedup vs baseline: 11.9308x; 11.9308x over previous
import functools

import jax
import jax.numpy as jnp
from jax import lax
from jax.experimental import pallas as pl
from jax.experimental.pallas import tpu as pltpu
from jax.experimental.pallas import tpu_sc as plsc

N_NODES = 10000
D_FEAT = 128

NC = 2
NS = 16
NW = NC * NS
CH = 128
SB = 8
LANES = 16

N_PAD = ((N_NODES + NS * CH - 1) // (NS * CH)) * NS * CH
RPS = N_PAD // NS


def _hist_body(idx_hbm, hist_hbm, ibuf0, ibuf1, ones_v, zbuf_v,
               hin_sh, hout_sh, isem0, isem1, *, nsb):
    c = lax.axis_index("c")
    s = lax.axis_index("s")
    wid = c * NS + s

    @pl.loop(0, RPS // LANES)
    def _zero(i):
        zbuf_v[pl.ds(i * LANES, LANES)] = jnp.zeros((LANES,), jnp.float32)

    @pl.loop(0, CH // LANES)
    def _ones(i):
        ones_v[pl.ds(i * LANES, LANES)] = jnp.ones((LANES,), jnp.float32)

    pltpu.sync_copy(zbuf_v, hin_sh.at[pl.ds(s * RPS, RPS)])
    pltpu.sync_copy(zbuf_v, hout_sh.at[pl.ds(s * RPS, RPS)])
    plsc.subcore_barrier()

    ibufs = (ibuf0, ibuf1)
    isems = (isem0, isem1)

    def fetch(si, q):
        pltpu.async_copy(idx_hbm.at[wid, si], ibufs[q], isems[q])

    def wait_fetch(si, q):
        pltpu.make_async_copy(idx_hbm.at[wid, si], ibufs[q], isems[q]).wait()

    def scatter_sc(q):
        for j in range(SB):
            pltpu.sync_copy(ones_v, hin_sh.at[ibufs[q].at[j, 1]], add=True)
            pltpu.sync_copy(ones_v, hout_sh.at[ibufs[q].at[j, 0]], add=True)

    fetch(0, 0)
    fetch(1, 1)

    @pl.loop(0, (nsb - 2) // 2)
    def _main(t):
        si = 2 * t
        wait_fetch(si, 0)
        scatter_sc(0)
        fetch(si + 2, 0)
        wait_fetch(si + 1, 1)
        scatter_sc(1)
        fetch(si + 3, 1)

    wait_fetch(nsb - 2, 0)
    scatter_sc(0)
    wait_fetch(nsb - 1, 1)
    scatter_sc(1)

    plsc.subcore_barrier()

    pltpu.sync_copy(hin_sh.at[pl.ds(s * RPS, RPS)], zbuf_v)
    pltpu.sync_copy(zbuf_v, hist_hbm.at[c, 0, pl.ds(s * RPS, RPS)])
    pltpu.sync_copy(hout_sh.at[pl.ds(s * RPS, RPS)], zbuf_v)
    pltpu.sync_copy(zbuf_v, hist_hbm.at[c, 1, pl.ds(s * RPS, RPS)])


def _agg_body(g_hbm, idx_hbm, p_hbm, ibuf0, ibuf1, row_a, row_b,
              acc_sh, isem0, isem1, gsem_a, gsem_b, *, nsb):
    c = lax.axis_index("c")
    s = lax.axis_index("s")
    wid = c * NS + s

    ibufs = (ibuf0, ibuf1)
    isems = (isem0, isem1)
    rows = (row_a, row_b)
    gsems = (gsem_a, gsem_b)

    @pl.loop(0, CH)
    def _zr(i):
        @pl.loop(0, D_FEAT // LANES)
        def _zc(j):
            row_a[i, pl.ds(j * LANES, LANES)] = jnp.zeros((LANES,), jnp.float32)

    for r in range(RPS // CH):
        pltpu.sync_copy(row_a, acc_sh.at[pl.ds(s * RPS + r * CH, CH)])
    plsc.subcore_barrier()

    def fetch(si, q):
        pltpu.async_copy(idx_hbm.at[wid, si], ibufs[q], isems[q])

    def wait_fetch(si, q):
        pltpu.make_async_copy(idx_hbm.at[wid, si], ibufs[q], isems[q]).wait()

    def start_gather(q, j, p):
        pltpu.async_copy(g_hbm.at[ibufs[q].at[j, 0]], rows[p], gsems[p])

    def wait_gather(q, j, p):
        pltpu.make_async_copy(g_hbm.at[ibufs[q].at[j, 0]], rows[p], gsems[p]).wait()

    def scatter(q, j, p):
        pltpu.sync_copy(rows[p], acc_sh.at[ibufs[q].at[j, 1]], add=True)

    def super_step(si, q, last_sc, tail_fetch):
        for j in range(SB - 1):
            start_gather(q, j + 1, (j + 1) % 2)
            wait_gather(q, j, j % 2)
            scatter(q, j, j % 2)
        if not last_sc:
            wait_fetch(si + 1, 1 - q)
            start_gather(1 - q, 0, 0)
        wait_gather(q, SB - 1, 1)
        scatter(q, SB - 1, 1)
        if tail_fetch:
            fetch(si + 2, q)

    pltpu.sync_copy(idx_hbm.at[wid, 0], ibuf0)
    fetch(1, 1)
    start_gather(0, 0, 0)

    @pl.loop(0, (nsb - 2) // 2)
    def _main(t):
        si = 2 * t
        super_step(si, 0, last_sc=False, tail_fetch=True)
        super_step(si + 1, 1, last_sc=False, tail_fetch=True)

    super_step(nsb - 2, 0, last_sc=False, tail_fetch=False)
    super_step(nsb - 1, 1, last_sc=True, tail_fetch=False)

    plsc.subcore_barrier()

    for r in range(RPS // CH):
        rr = pl.ds(s * RPS + r * CH, CH)
        pltpu.sync_copy(acc_sh.at[rr], row_a)
        pltpu.sync_copy(row_a, p_hbm.at[c, rr])


def _scale_body(hout_ref, h_ref, g_ref):
    deg = hout_ref[0, :] + hout_ref[1, :]
    f = jnp.where(deg > 0.0, lax.rsqrt(deg), 0.0)
    g_ref[...] = h_ref[...] * f[:, None]


def _combine_body(hin_ref, p_ref, o_ref):
    deg = hin_ref[0, :] + hin_ref[1, :]
    f = jnp.where(deg > 0.0, lax.rsqrt(deg), 0.0)
    o_ref[...] = (p_ref[0] + p_ref[1]) * f[:, None]


def kernel(node_embeddings, adjacency_list):
    n_edges = adjacency_list.shape[0]
    nsb = -(-n_edges // (NW * CH * SB))
    if nsb % 2:
        nsb += 1
    assert nsb >= 4
    k_chunks = nsb * SB
    e_pad = NW * k_chunks * CH

    src = adjacency_list[:, 0].astype(jnp.int32)
    tgt = adjacency_list[:, 1].astype(jnp.int32)
    pad = jnp.full((e_pad - n_edges,), N_NODES, jnp.int32)
    src3 = jnp.concatenate([src, pad]).reshape(NW, k_chunks, CH)
    tgt3 = jnp.concatenate([tgt, pad]).reshape(NW, k_chunks, CH)
    idx5 = jnp.stack([src3, tgt3], axis=2).reshape(NW, nsb, SB, 2, CH)
    h_pad = jnp.pad(node_embeddings, ((0, N_PAD - N_NODES), (0, 0)))

    mesh = plsc.VectorSubcoreMesh(core_axis_name="c", subcore_axis_name="s")

    hist = pl.kernel(
        functools.partial(_hist_body, nsb=nsb),
        out_type=jax.ShapeDtypeStruct((NC, 2, N_PAD), jnp.float32),
        mesh=mesh,
        scratch_types=[
            pltpu.VMEM((SB, 2, CH), jnp.int32),
            pltpu.VMEM((SB, 2, CH), jnp.int32),
            pltpu.VMEM((CH,), jnp.float32),
            pltpu.VMEM((RPS,), jnp.float32),
            pltpu.VMEM_SHARED((N_PAD,), jnp.float32),
            pltpu.VMEM_SHARED((N_PAD,), jnp.float32),
            pltpu.SemaphoreType.DMA,
            pltpu.SemaphoreType.DMA,
        ],
    )(idx5)

    blk = N_PAD // 8
    g = pl.pallas_call(
        _scale_body,
        grid=(8,),
        in_specs=[
            pl.BlockSpec((NC, blk), lambda i: (0, i)),
            pl.BlockSpec((blk, D_FEAT), lambda i: (i, 0)),
        ],
        out_specs=pl.BlockSpec((blk, D_FEAT), lambda i: (i, 0)),
        out_shape=jax.ShapeDtypeStruct((N_PAD, D_FEAT), jnp.float32),
    )(hist[:, 1, :], h_pad)

    partials = pl.kernel(
        functools.partial(_agg_body, nsb=nsb),
        out_type=jax.ShapeDtypeStruct((NC, N_PAD, D_FEAT), jnp.float32),
        mesh=mesh,
        scratch_types=[
            pltpu.VMEM((SB, 2, CH), jnp.int32),
            pltpu.VMEM((SB, 2, CH), jnp.int32),
            pltpu.VMEM((CH, D_FEAT), jnp.float32),
            pltpu.VMEM((CH, D_FEAT), jnp.float32),
            pltpu.VMEM_SHARED((N_PAD, D_FEAT), jnp.float32),
            pltpu.SemaphoreType.DMA,
            pltpu.SemaphoreType.DMA,
            pltpu.SemaphoreType.DMA,
            pltpu.SemaphoreType.DMA,
        ],
    )(g, idx5)

    out = pl.pallas_call(
        _combine_body,
        grid=(8,),
        in_specs=[
            pl.BlockSpec((NC, blk), lambda i: (0, i)),
            pl.BlockSpec((NC, blk, D_FEAT), lambda i: (0, i, 0)),
        ],
        out_specs=pl.BlockSpec((blk, D_FEAT), lambda i: (i, 0)),
        out_shape=jax.ShapeDtypeStruct((N_PAD, D_FEAT), jnp.float32),
    )(hist[:, 0, :], partials)

    return out[:N_NODES]


# speedup vs baseline: 35.7418x; 2.9958x over previous
import functools

import jax
import jax.numpy as jnp
from jax import lax
from jax.experimental import pallas as pl
from jax.experimental.pallas import tpu as pltpu
from jax.experimental.pallas import tpu_sc as plsc

N_NODES = 10000
D_FEAT = 128

NC = 2
NS = 16
NW = NC * NS
CH = 128
SB = 8
LANES = 16

N_PAD = ((N_NODES + NS * CH - 1) // (NS * CH)) * NS * CH
RPS = N_PAD // NS


def _hist_body(idx_hbm, hist_hbm, ibuf0, ibuf1, ones_v, zbuf_v,
               hin_sh, hout_sh, isem0, isem1, *, nsb):
    c = lax.axis_index("c")
    s = lax.axis_index("s")
    wid = c * NS + s

    @pl.loop(0, RPS // LANES)
    def _zero(i):
        zbuf_v[pl.ds(i * LANES, LANES)] = jnp.zeros((LANES,), jnp.float32)

    @pl.loop(0, CH // LANES)
    def _ones(i):
        ones_v[pl.ds(i * LANES, LANES)] = jnp.ones((LANES,), jnp.float32)

    pltpu.sync_copy(zbuf_v, hin_sh.at[pl.ds(s * RPS, RPS)])
    pltpu.sync_copy(zbuf_v, hout_sh.at[pl.ds(s * RPS, RPS)])
    plsc.subcore_barrier()

    ibufs = (ibuf0, ibuf1)
    isems = (isem0, isem1)

    def fetch(si, q):
        pltpu.async_copy(idx_hbm.at[wid, si], ibufs[q], isems[q])

    def wait_fetch(si, q):
        pltpu.make_async_copy(idx_hbm.at[wid, si], ibufs[q], isems[q]).wait()

    def scatter_sc(q):
        for j in range(SB):
            pltpu.sync_copy(ones_v, hin_sh.at[ibufs[q].at[j, 1]], add=True)
            pltpu.sync_copy(ones_v, hout_sh.at[ibufs[q].at[j, 0]], add=True)

    fetch(0, 0)
    fetch(1, 1)

    @pl.loop(0, (nsb - 2) // 2)
    def _main(t):
        si = 2 * t
        wait_fetch(si, 0)
        scatter_sc(0)
        fetch(si + 2, 0)
        wait_fetch(si + 1, 1)
        scatter_sc(1)
        fetch(si + 3, 1)

    wait_fetch(nsb - 2, 0)
    scatter_sc(0)
    wait_fetch(nsb - 1, 1)
    scatter_sc(1)

    plsc.subcore_barrier()

    pltpu.sync_copy(hin_sh.at[pl.ds(s * RPS, RPS)], zbuf_v)
    pltpu.sync_copy(zbuf_v, hist_hbm.at[c, 0, pl.ds(s * RPS, RPS)])
    pltpu.sync_copy(hout_sh.at[pl.ds(s * RPS, RPS)], zbuf_v)
    pltpu.sync_copy(zbuf_v, hist_hbm.at[c, 1, pl.ds(s * RPS, RPS)])


def _agg_body(g_hbm, idx_hbm, p_hbm, ibuf0, ibuf1, row_a, row_b,
              acc_sh, isem0, isem1, gsem_a, gsem_b, *, nsb):
    c = lax.axis_index("c")
    s = lax.axis_index("s")
    wid = c * NS + s

    ibufs = (ibuf0, ibuf1)
    isems = (isem0, isem1)
    rows = (row_a, row_b)
    gsems = (gsem_a, gsem_b)

    @pl.loop(0, CH)
    def _zr(i):
        @pl.loop(0, D_FEAT // LANES)
        def _zc(j):
            row_a[i, pl.ds(j * LANES, LANES)] = jnp.zeros((LANES,), jnp.float32)

    for r in range(RPS // CH):
        pltpu.sync_copy(row_a, acc_sh.at[pl.ds(s * RPS + r * CH, CH)])
    plsc.subcore_barrier()

    def fetch(si, q):
        pltpu.async_copy(idx_hbm.at[wid, si], ibufs[q], isems[q])

    def wait_fetch(si, q):
        pltpu.make_async_copy(idx_hbm.at[wid, si], ibufs[q], isems[q]).wait()

    def start_gather(q, j, p):
        pltpu.async_copy(g_hbm.at[ibufs[q].at[j, 0]], rows[p], gsems[p])

    def wait_gather(q, j, p):
        pltpu.make_async_copy(g_hbm.at[ibufs[q].at[j, 0]], rows[p], gsems[p]).wait()

    def scatter(q, j, p):
        pltpu.sync_copy(rows[p], acc_sh.at[ibufs[q].at[j, 1]], add=True)

    def super_step(si, q, last_sc, tail_fetch):
        for j in range(SB - 1):
            start_gather(q, j + 1, (j + 1) % 2)
            wait_gather(q, j, j % 2)
            scatter(q, j, j % 2)
        if not last_sc:
            wait_fetch(si + 1, 1 - q)
            start_gather(1 - q, 0, 0)
        wait_gather(q, SB - 1, 1)
        scatter(q, SB - 1, 1)
        if tail_fetch:
            fetch(si + 2, q)

    pltpu.sync_copy(idx_hbm.at[wid, 0], ibuf0)
    fetch(1, 1)
    start_gather(0, 0, 0)

    @pl.loop(0, (nsb - 2) // 2)
    def _main(t):
        si = 2 * t
        super_step(si, 0, last_sc=False, tail_fetch=True)
        super_step(si + 1, 1, last_sc=False, tail_fetch=True)

    super_step(nsb - 2, 0, last_sc=False, tail_fetch=False)
    super_step(nsb - 1, 1, last_sc=True, tail_fetch=False)

    plsc.subcore_barrier()

    for r in range(RPS // CH):
        rr = pl.ds(s * RPS + r * CH, CH)
        pltpu.sync_copy(acc_sh.at[rr], row_a)
        pltpu.sync_copy(row_a, p_hbm.at[c, rr])


def _scale_body(hout_ref, h_ref, g_ref):
    deg = hout_ref[0, :] + hout_ref[1, :]
    f = jnp.where(deg > 0.0, lax.rsqrt(deg), 0.0)
    g_ref[...] = h_ref[...] * f[:, None]


def _combine_body(hin_ref, p_ref, o_ref):
    deg = hin_ref[0, :] + hin_ref[1, :]
    f = jnp.where(deg > 0.0, lax.rsqrt(deg), 0.0)
    o_ref[...] = (p_ref[0] + p_ref[1]) * f[:, None]


def kernel(node_embeddings, adjacency_list):
    n_edges = adjacency_list.shape[0]
    nsb = -(-n_edges // (NW * CH * SB))
    if nsb % 2:
        nsb += 1
    assert nsb >= 4
    k_chunks = nsb * SB
    e_pad = NW * k_chunks * CH

    src = adjacency_list[:, 0].astype(jnp.int32)
    tgt = adjacency_list[:, 1].astype(jnp.int32)
    pad = N_NODES + (jnp.arange(e_pad - n_edges, dtype=jnp.int32)
                     % (N_PAD - N_NODES))
    src3 = jnp.concatenate([src, pad]).reshape(NW, k_chunks, CH)
    tgt3 = jnp.concatenate([tgt, pad]).reshape(NW, k_chunks, CH)
    idx5 = jnp.stack([src3, tgt3], axis=2).reshape(NW, nsb, SB, 2, CH)
    h_pad = jnp.pad(node_embeddings, ((0, N_PAD - N_NODES), (0, 0)))

    mesh = plsc.VectorSubcoreMesh(core_axis_name="c", subcore_axis_name="s")

    hist = pl.kernel(
        functools.partial(_hist_body, nsb=nsb),
        out_type=jax.ShapeDtypeStruct((NC, 2, N_PAD), jnp.float32),
        mesh=mesh,
        scratch_types=[
            pltpu.VMEM((SB, 2, CH), jnp.int32),
            pltpu.VMEM((SB, 2, CH), jnp.int32),
            pltpu.VMEM((CH,), jnp.float32),
            pltpu.VMEM((RPS,), jnp.float32),
            pltpu.VMEM_SHARED((N_PAD,), jnp.float32),
            pltpu.VMEM_SHARED((N_PAD,), jnp.float32),
            pltpu.SemaphoreType.DMA,
            pltpu.SemaphoreType.DMA,
        ],
    )(idx5)

    blk = N_PAD // 8
    g = pl.pallas_call(
        _scale_body,
        grid=(8,),
        in_specs=[
            pl.BlockSpec((NC, blk), lambda i: (0, i)),
            pl.BlockSpec((blk, D_FEAT), lambda i: (i, 0)),
        ],
        out_specs=pl.BlockSpec((blk, D_FEAT), lambda i: (i, 0)),
        out_shape=jax.ShapeDtypeStruct((N_PAD, D_FEAT), jnp.float32),
    )(hist[:, 1, :], h_pad)

    partials = pl.kernel(
        functools.partial(_agg_body, nsb=nsb),
        out_type=jax.ShapeDtypeStruct((NC, N_PAD, D_FEAT), jnp.float32),
        mesh=mesh,
        scratch_types=[
            pltpu.VMEM((SB, 2, CH), jnp.int32),
            pltpu.VMEM((SB, 2, CH), jnp.int32),
            pltpu.VMEM((CH, D_FEAT), jnp.float32),
            pltpu.VMEM((CH, D_FEAT), jnp.float32),
            pltpu.VMEM_SHARED((N_PAD, D_FEAT), jnp.float32),
            pltpu.SemaphoreType.DMA,
            pltpu.SemaphoreType.DMA,
            pltpu.SemaphoreType.DMA,
            pltpu.SemaphoreType.DMA,
        ],
    )(g, idx5)

    out = pl.pallas_call(
        _combine_body,
        grid=(8,),
        in_specs=[
            pl.BlockSpec((NC, blk), lambda i: (0, i)),
            pl.BlockSpec((NC, blk, D_FEAT), lambda i: (0, i, 0)),
        ],
        out_specs=pl.BlockSpec((blk, D_FEAT), lambda i: (i, 0)),
        out_shape=jax.ShapeDtypeStruct((N_PAD, D_FEAT), jnp.float32),
    )(hist[:, 0, :], partials)

    return out[:N_NODES]


# speedup vs baseline: 37.0327x; 1.0361x over previous
import functools

import jax
import jax.numpy as jnp
from jax import lax
from jax.experimental import pallas as pl
from jax.experimental.pallas import tpu as pltpu
from jax.experimental.pallas import tpu_sc as plsc

N_NODES = 10000
D_FEAT = 128

NC = 2
NS = 16
NW = NC * NS
CH = 128
SB = 8
LANES = 16

N_PAD = ((N_NODES + NS * CH - 1) // (NS * CH)) * NS * CH
RPS = N_PAD // NS


def _hist_body(idx_hbm, hist_hbm, ibuf0, ibuf1, ones_v, zbuf_v,
               hin_sh, hout_sh, isem0, isem1, *, nsb):
    c = lax.axis_index("c")
    s = lax.axis_index("s")
    wid = c * NS + s

    @pl.loop(0, RPS // LANES)
    def _zero(i):
        zbuf_v[pl.ds(i * LANES, LANES)] = jnp.zeros((LANES,), jnp.float32)

    @pl.loop(0, CH // LANES)
    def _ones(i):
        ones_v[pl.ds(i * LANES, LANES)] = jnp.ones((LANES,), jnp.float32)

    pltpu.sync_copy(zbuf_v, hin_sh.at[pl.ds(s * RPS, RPS)])
    pltpu.sync_copy(zbuf_v, hout_sh.at[pl.ds(s * RPS, RPS)])
    plsc.subcore_barrier()

    ibufs = (ibuf0, ibuf1)
    isems = (isem0, isem1)

    def fetch(si, q):
        pltpu.async_copy(idx_hbm.at[wid, si], ibufs[q], isems[q])

    def wait_fetch(si, q):
        pltpu.make_async_copy(idx_hbm.at[wid, si], ibufs[q], isems[q]).wait()

    def scatter_sc(q):
        for j in range(SB):
            pltpu.sync_copy(ones_v, hin_sh.at[ibufs[q].at[j, 1]], add=True)
            pltpu.sync_copy(ones_v, hout_sh.at[ibufs[q].at[j, 0]], add=True)

    fetch(0, 0)
    fetch(1, 1)

    @pl.loop(0, (nsb - 2) // 2)
    def _main(t):
        si = 2 * t
        wait_fetch(si, 0)
        scatter_sc(0)
        fetch(si + 2, 0)
        wait_fetch(si + 1, 1)
        scatter_sc(1)
        fetch(si + 3, 1)

    wait_fetch(nsb - 2, 0)
    scatter_sc(0)
    wait_fetch(nsb - 1, 1)
    scatter_sc(1)

    plsc.subcore_barrier()

    pltpu.sync_copy(hin_sh.at[pl.ds(s * RPS, RPS)], zbuf_v)
    pltpu.sync_copy(zbuf_v, hist_hbm.at[c, 0, pl.ds(s * RPS, RPS)])
    pltpu.sync_copy(hout_sh.at[pl.ds(s * RPS, RPS)], zbuf_v)
    pltpu.sync_copy(zbuf_v, hist_hbm.at[c, 1, pl.ds(s * RPS, RPS)])


def _agg_body(g_hbm, idx_hbm, p_hbm, ibuf0, ibuf1, row_a, row_b,
              acc_sh, isem0, isem1, gsem_a, gsem_b, *, nsb):
    c = lax.axis_index("c")
    s = lax.axis_index("s")
    wid = c * NS + s

    ibufs = (ibuf0, ibuf1)
    isems = (isem0, isem1)
    rows = (row_a, row_b)
    gsems = (gsem_a, gsem_b)

    @pl.loop(0, CH)
    def _zr(i):
        @pl.loop(0, D_FEAT // LANES)
        def _zc(j):
            row_a[i, pl.ds(j * LANES, LANES)] = jnp.zeros((LANES,), jnp.float32)

    for r in range(RPS // CH):
        pltpu.sync_copy(row_a, acc_sh.at[pl.ds(s * RPS + r * CH, CH)])
    plsc.subcore_barrier()

    def fetch(si, q):
        pltpu.async_copy(idx_hbm.at[wid, si], ibufs[q], isems[q])

    def wait_fetch(si, q):
        pltpu.make_async_copy(idx_hbm.at[wid, si], ibufs[q], isems[q]).wait()

    def start_gather(q, j, p):
        pltpu.async_copy(g_hbm.at[ibufs[q].at[j, 0]], rows[p], gsems[p])

    def wait_gather(q, j, p):
        pltpu.make_async_copy(g_hbm.at[ibufs[q].at[j, 0]], rows[p], gsems[p]).wait()

    def scatter(q, j, p):
        pltpu.sync_copy(rows[p], acc_sh.at[ibufs[q].at[j, 1]], add=True)

    def super_step(si, q, last_sc, tail_fetch):
        for j in range(SB - 1):
            start_gather(q, j + 1, (j + 1) % 2)
            wait_gather(q, j, j % 2)
            scatter(q, j, j % 2)
        if not last_sc:
            wait_fetch(si + 1, 1 - q)
            start_gather(1 - q, 0, 0)
        wait_gather(q, SB - 1, 1)
        scatter(q, SB - 1, 1)
        if tail_fetch:
            fetch(si + 2, q)

    pltpu.sync_copy(idx_hbm.at[wid, 0], ibuf0)
    fetch(1, 1)
    start_gather(0, 0, 0)

    @pl.loop(0, (nsb - 2) // 2)
    def _main(t):
        si = 2 * t
        super_step(si, 0, last_sc=False, tail_fetch=True)
        super_step(si + 1, 1, last_sc=False, tail_fetch=True)

    super_step(nsb - 2, 0, last_sc=False, tail_fetch=False)
    super_step(nsb - 1, 1, last_sc=True, tail_fetch=False)

    plsc.subcore_barrier()

    for r in range(RPS // CH):
        rr = pl.ds(s * RPS + r * CH, CH)
        pltpu.sync_copy(acc_sh.at[rr], row_a)
        pltpu.sync_copy(row_a, p_hbm.at[c, rr])


def _scale_body(hist_ref, h_ref, g_ref, *, blk):
    i = pl.program_id(0)
    deg = hist_ref[0, 1, :] + hist_ref[1, 1, :]
    f = jnp.where(deg > 0.0, lax.rsqrt(deg), 0.0)
    row = i * blk + lax.broadcasted_iota(jnp.int32, (blk, 1), 0)
    g_ref[...] = jnp.where(row < N_NODES, h_ref[...] * f[:, None], 0.0)


def _combine_body(hist_ref, p_ref, o_ref):
    deg = hist_ref[0, 0, :] + hist_ref[1, 0, :]
    f = jnp.where(deg > 0.0, lax.rsqrt(deg), 0.0)
    o_ref[...] = (p_ref[0] + p_ref[1]) * f[:, None]


def kernel(node_embeddings, adjacency_list):
    n_edges = adjacency_list.shape[0]
    nsb = -(-n_edges // (NW * CH * SB))
    if nsb % 2:
        nsb += 1
    assert nsb >= 4
    k_chunks = nsb * SB
    e_pad = NW * k_chunks * CH

    src = adjacency_list[:, 0].astype(jnp.int32)
    tgt = adjacency_list[:, 1].astype(jnp.int32)
    pad = N_NODES + (jnp.arange(e_pad - n_edges, dtype=jnp.int32)
                     % (N_PAD - N_NODES))
    src3 = jnp.concatenate([src, pad]).reshape(NW, k_chunks, CH)
    tgt3 = jnp.concatenate([tgt, pad]).reshape(NW, k_chunks, CH)
    idx5 = jnp.stack([src3, tgt3], axis=2).reshape(NW, nsb, SB, 2, CH)

    mesh = plsc.VectorSubcoreMesh(core_axis_name="c", subcore_axis_name="s")

    hist = pl.kernel(
        functools.partial(_hist_body, nsb=nsb),
        out_type=jax.ShapeDtypeStruct((NC, 2, N_PAD), jnp.float32),
        mesh=mesh,
        scratch_types=[
            pltpu.VMEM((SB, 2, CH), jnp.int32),
            pltpu.VMEM((SB, 2, CH), jnp.int32),
            pltpu.VMEM((CH,), jnp.float32),
            pltpu.VMEM((RPS,), jnp.float32),
            pltpu.VMEM_SHARED((N_PAD,), jnp.float32),
            pltpu.VMEM_SHARED((N_PAD,), jnp.float32),
            pltpu.SemaphoreType.DMA,
            pltpu.SemaphoreType.DMA,
        ],
    )(idx5)

    blk = N_PAD // 8
    g = pl.pallas_call(
        functools.partial(_scale_body, blk=blk),
        grid=(8,),
        in_specs=[
            pl.BlockSpec((NC, 2, blk), lambda i: (0, 0, i)),
            pl.BlockSpec((blk, D_FEAT), lambda i: (i, 0)),
        ],
        out_specs=pl.BlockSpec((blk, D_FEAT), lambda i: (i, 0)),
        out_shape=jax.ShapeDtypeStruct((N_PAD, D_FEAT), jnp.float32),
    )(hist, node_embeddings)

    partials = pl.kernel(
        functools.partial(_agg_body, nsb=nsb),
        out_type=jax.ShapeDtypeStruct((NC, N_PAD, D_FEAT), jnp.float32),
        mesh=mesh,
        scratch_types=[
            pltpu.VMEM((SB, 2, CH), jnp.int32),
            pltpu.VMEM((SB, 2, CH), jnp.int32),
            pltpu.VMEM((CH, D_FEAT), jnp.float32),
            pltpu.VMEM((CH, D_FEAT), jnp.float32),
            pltpu.VMEM_SHARED((N_PAD, D_FEAT), jnp.float32),
            pltpu.SemaphoreType.DMA,
            pltpu.SemaphoreType.DMA,
            pltpu.SemaphoreType.DMA,
            pltpu.SemaphoreType.DMA,
        ],
    )(g, idx5)

    out = pl.pallas_call(
        _combine_body,
        grid=(8,),
        in_specs=[
            pl.BlockSpec((NC, 2, blk), lambda i: (0, 0, i)),
            pl.BlockSpec((NC, blk, D_FEAT), lambda i: (0, i, 0)),
        ],
        out_specs=pl.BlockSpec((blk, D_FEAT), lambda i: (i, 0)),
        out_shape=jax.ShapeDtypeStruct((N_NODES, D_FEAT), jnp.float32),
    )(hist, partials)

    return out


# speedup vs baseline: 39.5654x; 1.0684x over previous
import functools

import jax
import jax.numpy as jnp
from jax import lax
from jax.experimental import pallas as pl
from jax.experimental.pallas import tpu as pltpu
from jax.experimental.pallas import tpu_sc as plsc

N_NODES = 10000
D_FEAT = 128

NC = 2
NS = 16
NW = NC * NS
CH = 128
SB = 8
LANES = 16

N_PAD = ((N_NODES + NS * CH - 1) // (NS * CH)) * NS * CH
RPS = N_PAD // NS


def _hist_body(idx_hbm, hist_hbm, ibuf0, ibuf1, ones_v, zbuf_v,
               hin_sh, hout_sh, isem0, isem1, *, nsb):
    c = lax.axis_index("c")
    s = lax.axis_index("s")
    wid = c * NS + s

    @pl.loop(0, RPS // LANES)
    def _zero(i):
        zbuf_v[pl.ds(i * LANES, LANES)] = jnp.zeros((LANES,), jnp.float32)

    @pl.loop(0, CH // LANES)
    def _ones(i):
        ones_v[pl.ds(i * LANES, LANES)] = jnp.ones((LANES,), jnp.float32)

    pltpu.sync_copy(zbuf_v, hin_sh.at[pl.ds(s * RPS, RPS)])
    pltpu.sync_copy(zbuf_v, hout_sh.at[pl.ds(s * RPS, RPS)])
    plsc.subcore_barrier()

    ibufs = (ibuf0, ibuf1)
    isems = (isem0, isem1)

    def fetch(si, q):
        pltpu.async_copy(idx_hbm.at[wid, si], ibufs[q], isems[q])

    def wait_fetch(si, q):
        pltpu.make_async_copy(idx_hbm.at[wid, si], ibufs[q], isems[q]).wait()

    def scatter_sc(q):
        for j in range(SB):
            pltpu.sync_copy(ones_v, hin_sh.at[ibufs[q].at[j, 1]], add=True)
            pltpu.sync_copy(ones_v, hout_sh.at[ibufs[q].at[j, 0]], add=True)

    fetch(0, 0)
    fetch(1, 1)

    @pl.loop(0, (nsb - 2) // 2)
    def _main(t):
        si = 2 * t
        wait_fetch(si, 0)
        scatter_sc(0)
        fetch(si + 2, 0)
        wait_fetch(si + 1, 1)
        scatter_sc(1)
        fetch(si + 3, 1)

    wait_fetch(nsb - 2, 0)
    scatter_sc(0)
    wait_fetch(nsb - 1, 1)
    scatter_sc(1)

    plsc.subcore_barrier()

    pltpu.sync_copy(hin_sh.at[pl.ds(s * RPS, RPS)], zbuf_v)
    pltpu.sync_copy(zbuf_v, hist_hbm.at[c, 0, pl.ds(s * RPS, RPS)])
    pltpu.sync_copy(hout_sh.at[pl.ds(s * RPS, RPS)], zbuf_v)
    pltpu.sync_copy(zbuf_v, hist_hbm.at[c, 1, pl.ds(s * RPS, RPS)])


def _agg_body(g_hbm, idx_hbm, p_hbm, ibuf0, ibuf1, row_a, row_b,
              acc_sh, isem0, isem1, gsem_a, gsem_b, *, nsb):
    c = lax.axis_index("c")
    s = lax.axis_index("s")
    wid = c * NS + s

    ibufs = (ibuf0, ibuf1)
    isems = (isem0, isem1)
    rows = (row_a, row_b)
    gsems = (gsem_a, gsem_b)

    @pl.loop(0, CH)
    def _zr(i):
        @pl.loop(0, D_FEAT // LANES)
        def _zc(j):
            row_a[i, pl.ds(j * LANES, LANES)] = jnp.zeros((LANES,), jnp.float32)

    for r in range(RPS // CH):
        pltpu.sync_copy(row_a, acc_sh.at[pl.ds(s * RPS + r * CH, CH)])
    plsc.subcore_barrier()

    def fetch(si, q):
        pltpu.async_copy(idx_hbm.at[wid, si], ibufs[q], isems[q])

    def wait_fetch(si, q):
        pltpu.make_async_copy(idx_hbm.at[wid, si], ibufs[q], isems[q]).wait()

    def start_gather(q, j, p):
        pltpu.async_copy(g_hbm.at[ibufs[q].at[j, 0]], rows[p], gsems[p])

    def wait_gather(q, j, p):
        pltpu.make_async_copy(g_hbm.at[ibufs[q].at[j, 0]], rows[p], gsems[p]).wait()

    def scatter(q, j, p):
        pltpu.sync_copy(rows[p], acc_sh.at[ibufs[q].at[j, 1]], add=True)

    def super_step(si, q, last_sc, tail_fetch):
        for j in range(SB - 1):
            start_gather(q, j + 1, (j + 1) % 2)
            wait_gather(q, j, j % 2)
            scatter(q, j, j % 2)
        if not last_sc:
            wait_fetch(si + 1, 1 - q)
            start_gather(1 - q, 0, 0)
        wait_gather(q, SB - 1, 1)
        scatter(q, SB - 1, 1)
        if tail_fetch:
            fetch(si + 2, q)

    pltpu.sync_copy(idx_hbm.at[wid, 0], ibuf0)
    fetch(1, 1)
    start_gather(0, 0, 0)

    @pl.loop(0, (nsb - 2) // 2)
    def _main(t):
        si = 2 * t
        super_step(si, 0, last_sc=False, tail_fetch=True)
        super_step(si + 1, 1, last_sc=False, tail_fetch=True)

    super_step(nsb - 2, 0, last_sc=False, tail_fetch=False)
    super_step(nsb - 1, 1, last_sc=True, tail_fetch=False)

    plsc.subcore_barrier()

    for r in range(RPS // CH):
        rr = pl.ds(s * RPS + r * CH, CH)
        pltpu.sync_copy(acc_sh.at[rr], row_a)
        pltpu.sync_copy(row_a, p_hbm.at[c, rr])


def _scale_body(hist_ref, h_ref, g_ref, *, blk):
    i = pl.program_id(0)
    deg = hist_ref[0, 1, :] + hist_ref[1, 1, :]
    f = jnp.where(deg > 0.0, lax.rsqrt(deg), 0.0)
    row = i * blk + lax.broadcasted_iota(jnp.int32, (blk, 1), 0)
    g_ref[...] = jnp.where(row < N_NODES, h_ref[...] * f[:, None], 0.0)


def _combine_body(hist_ref, p_ref, o_ref):
    deg = hist_ref[0, 0, :] + hist_ref[1, 0, :]
    f = jnp.where(deg > 0.0, lax.rsqrt(deg), 0.0)
    o_ref[...] = (p_ref[0] + p_ref[1]) * f[:, None]


def kernel(node_embeddings, adjacency_list):
    n_edges = adjacency_list.shape[0]
    nsb = -(-n_edges // (NW * CH * SB))
    if nsb % 2:
        nsb += 1
    assert nsb >= 4
    k_chunks = nsb * SB
    e_pad = NW * k_chunks * CH

    pad = N_NODES + (jnp.arange(e_pad - n_edges, dtype=jnp.int32)
                     % (N_PAD - N_NODES))
    ap = jnp.concatenate(
        [adjacency_list.astype(jnp.int32),
         jnp.broadcast_to(pad[:, None], (e_pad - n_edges, 2))])
    idx5 = jnp.swapaxes(ap.reshape(NW, nsb, SB, CH, 2), 3, 4)

    mesh = plsc.VectorSubcoreMesh(core_axis_name="c", subcore_axis_name="s")

    hist = pl.kernel(
        functools.partial(_hist_body, nsb=nsb),
        out_type=jax.ShapeDtypeStruct((NC, 2, N_PAD), jnp.float32),
        mesh=mesh,
        scratch_types=[
            pltpu.VMEM((SB, 2, CH), jnp.int32),
            pltpu.VMEM((SB, 2, CH), jnp.int32),
            pltpu.VMEM((CH,), jnp.float32),
            pltpu.VMEM((RPS,), jnp.float32),
            pltpu.VMEM_SHARED((N_PAD,), jnp.float32),
            pltpu.VMEM_SHARED((N_PAD,), jnp.float32),
            pltpu.SemaphoreType.DMA,
            pltpu.SemaphoreType.DMA,
        ],
    )(idx5)

    blk = N_PAD // 8
    g = pl.pallas_call(
        functools.partial(_scale_body, blk=blk),
        grid=(8,),
        in_specs=[
            pl.BlockSpec((NC, 2, blk), lambda i: (0, 0, i)),
            pl.BlockSpec((blk, D_FEAT), lambda i: (i, 0)),
        ],
        out_specs=pl.BlockSpec((blk, D_FEAT), lambda i: (i, 0)),
        out_shape=jax.ShapeDtypeStruct((N_PAD, D_FEAT), jnp.float32),
    )(hist, node_embeddings)

    partials = pl.kernel(
        functools.partial(_agg_body, nsb=nsb),
        out_type=jax.ShapeDtypeStruct((NC, N_PAD, D_FEAT), jnp.float32),
        mesh=mesh,
        scratch_types=[
            pltpu.VMEM((SB, 2, CH), jnp.int32),
            pltpu.VMEM((SB, 2, CH), jnp.int32),
            pltpu.VMEM((CH, D_FEAT), jnp.float32),
            pltpu.VMEM((CH, D_FEAT), jnp.float32),
            pltpu.VMEM_SHARED((N_PAD, D_FEAT), jnp.float32),
            pltpu.SemaphoreType.DMA,
            pltpu.SemaphoreType.DMA,
            pltpu.SemaphoreType.DMA,
            pltpu.SemaphoreType.DMA,
        ],
    )(g, idx5)

    out = pl.pallas_call(
        _combine_body,
        grid=(8,),
        in_specs=[
            pl.BlockSpec((NC, 2, blk), lambda i: (0, 0, i)),
            pl.BlockSpec((NC, blk, D_FEAT), lambda i: (0, i, 0)),
        ],
        out_specs=pl.BlockSpec((blk, D_FEAT), lambda i: (i, 0)),
        out_shape=jax.ShapeDtypeStruct((N_NODES, D_FEAT), jnp.float32),
    )(hist, partials)

    return out


# speedup vs baseline: 39.6630x; 1.0025x over previous
import functools

import jax
import jax.numpy as jnp
from jax import lax
from jax.experimental import pallas as pl
from jax.experimental.pallas import tpu as pltpu
from jax.experimental.pallas import tpu_sc as plsc

N_NODES = 10000
D_FEAT = 128

NC = 2
NS = 16
NW = NC * NS
CH = 128
SB = 8
LANES = 16

N_PAD = ((N_NODES + NS * CH - 1) // (NS * CH)) * NS * CH
RPS = N_PAD // NS


def _hist_body(idx_hbm, hist_hbm, ibuf0, ibuf1, ones_v, zbuf_v,
               hin_sh, hout_sh, isem0, isem1, *, nsb):
    c = lax.axis_index("c")
    s = lax.axis_index("s")
    wid = c * NS + s

    @pl.loop(0, RPS // LANES)
    def _zero(i):
        zbuf_v[pl.ds(i * LANES, LANES)] = jnp.zeros((LANES,), jnp.float32)

    @pl.loop(0, CH // LANES)
    def _ones(i):
        ones_v[pl.ds(i * LANES, LANES)] = jnp.ones((LANES,), jnp.float32)

    pltpu.sync_copy(zbuf_v, hin_sh.at[pl.ds(s * RPS, RPS)])
    pltpu.sync_copy(zbuf_v, hout_sh.at[pl.ds(s * RPS, RPS)])
    plsc.subcore_barrier()

    ibufs = (ibuf0, ibuf1)
    isems = (isem0, isem1)

    def fetch(si, q):
        pltpu.async_copy(idx_hbm.at[wid, si], ibufs[q], isems[q])

    def wait_fetch(si, q):
        pltpu.make_async_copy(idx_hbm.at[wid, si], ibufs[q], isems[q]).wait()

    def scatter_sc(q):
        for j in range(SB):
            pltpu.sync_copy(ones_v, hin_sh.at[ibufs[q].at[j, 1]], add=True)
            pltpu.sync_copy(ones_v, hout_sh.at[ibufs[q].at[j, 0]], add=True)

    fetch(0, 0)
    fetch(1, 1)

    @pl.loop(0, (nsb - 2) // 2)
    def _main(t):
        si = 2 * t
        wait_fetch(si, 0)
        scatter_sc(0)
        fetch(si + 2, 0)
        wait_fetch(si + 1, 1)
        scatter_sc(1)
        fetch(si + 3, 1)

    wait_fetch(nsb - 2, 0)
    scatter_sc(0)
    wait_fetch(nsb - 1, 1)
    scatter_sc(1)

    plsc.subcore_barrier()

    pltpu.sync_copy(hin_sh.at[pl.ds(s * RPS, RPS)], zbuf_v)
    pltpu.sync_copy(zbuf_v, hist_hbm.at[c, 0, pl.ds(s * RPS, RPS)])
    pltpu.sync_copy(hout_sh.at[pl.ds(s * RPS, RPS)], zbuf_v)
    pltpu.sync_copy(zbuf_v, hist_hbm.at[c, 1, pl.ds(s * RPS, RPS)])


def _agg_body(g_hbm, idx_hbm, p_hbm, ibuf0, ibuf1, row_a, row_b,
              acc_sh, isem0, isem1, gsem_a, gsem_b, *, nsb):
    c = lax.axis_index("c")
    s = lax.axis_index("s")
    wid = c * NS + s

    ibufs = (ibuf0, ibuf1)
    isems = (isem0, isem1)
    rows = (row_a, row_b)
    gsems = (gsem_a, gsem_b)

    @pl.loop(0, CH)
    def _zr(i):
        @pl.loop(0, D_FEAT // LANES)
        def _zc(j):
            row_a[i, pl.ds(j * LANES, LANES)] = jnp.zeros((LANES,), jnp.float32)

    for r in range(RPS // CH):
        pltpu.async_copy(row_a, acc_sh.at[pl.ds(s * RPS + r * CH, CH)], gsem_a)
    for r in range(RPS // CH):
        pltpu.make_async_copy(
            row_a, acc_sh.at[pl.ds(s * RPS + r * CH, CH)], gsem_a).wait()
    plsc.subcore_barrier()

    def fetch(si, q):
        pltpu.async_copy(idx_hbm.at[wid, si], ibufs[q], isems[q])

    def wait_fetch(si, q):
        pltpu.make_async_copy(idx_hbm.at[wid, si], ibufs[q], isems[q]).wait()

    def start_gather(q, j, p):
        pltpu.async_copy(g_hbm.at[ibufs[q].at[j, 0]], rows[p], gsems[p])

    def wait_gather(q, j, p):
        pltpu.make_async_copy(g_hbm.at[ibufs[q].at[j, 0]], rows[p], gsems[p]).wait()

    def scatter(q, j, p):
        pltpu.sync_copy(rows[p], acc_sh.at[ibufs[q].at[j, 1]], add=True)

    def super_step(si, q, last_sc, tail_fetch):
        for j in range(SB - 1):
            start_gather(q, j + 1, (j + 1) % 2)
            wait_gather(q, j, j % 2)
            scatter(q, j, j % 2)
        if not last_sc:
            wait_fetch(si + 1, 1 - q)
            start_gather(1 - q, 0, 0)
        wait_gather(q, SB - 1, 1)
        scatter(q, SB - 1, 1)
        if tail_fetch:
            fetch(si + 2, q)

    pltpu.sync_copy(idx_hbm.at[wid, 0], ibuf0)
    fetch(1, 1)
    start_gather(0, 0, 0)

    @pl.loop(0, (nsb - 2) // 2)
    def _main(t):
        si = 2 * t
        super_step(si, 0, last_sc=False, tail_fetch=True)
        super_step(si + 1, 1, last_sc=False, tail_fetch=True)

    super_step(nsb - 2, 0, last_sc=False, tail_fetch=False)
    super_step(nsb - 1, 1, last_sc=True, tail_fetch=False)

    plsc.subcore_barrier()

    rr = pl.ds(s * RPS, RPS)
    pltpu.sync_copy(acc_sh.at[rr], p_hbm.at[c, rr])


def _scale_body(hist_ref, h_ref, g_ref, *, blk):
    i = pl.program_id(0)
    deg = hist_ref[0, 1, :] + hist_ref[1, 1, :]
    f = jnp.where(deg > 0.0, lax.rsqrt(deg), 0.0)
    row = i * blk + lax.broadcasted_iota(jnp.int32, (blk, 1), 0)
    g_ref[...] = jnp.where(row < N_NODES, h_ref[...] * f[:, None], 0.0)


def _combine_body(hist_ref, p_ref, o_ref):
    deg = hist_ref[0, 0, :] + hist_ref[1, 0, :]
    f = jnp.where(deg > 0.0, lax.rsqrt(deg), 0.0)
    o_ref[...] = (p_ref[0] + p_ref[1]) * f[:, None]


def kernel(node_embeddings, adjacency_list):
    n_edges = adjacency_list.shape[0]
    nsb = -(-n_edges // (NW * CH * SB))
    if nsb % 2:
        nsb += 1
    assert nsb >= 4
    k_chunks = nsb * SB
    e_pad = NW * k_chunks * CH

    pad = N_NODES + (jnp.arange(e_pad - n_edges, dtype=jnp.int32)
                     % (N_PAD - N_NODES))
    ap = jnp.concatenate(
        [adjacency_list.astype(jnp.int32),
         jnp.broadcast_to(pad[:, None], (e_pad - n_edges, 2))])
    idx5 = jnp.swapaxes(ap.reshape(NW, nsb, SB, CH, 2), 3, 4)

    mesh = plsc.VectorSubcoreMesh(core_axis_name="c", subcore_axis_name="s")

    hist = pl.kernel(
        functools.partial(_hist_body, nsb=nsb),
        out_type=jax.ShapeDtypeStruct((NC, 2, N_PAD), jnp.float32),
        mesh=mesh,
        scratch_types=[
            pltpu.VMEM((SB, 2, CH), jnp.int32),
            pltpu.VMEM((SB, 2, CH), jnp.int32),
            pltpu.VMEM((CH,), jnp.float32),
            pltpu.VMEM((RPS,), jnp.float32),
            pltpu.VMEM_SHARED((N_PAD,), jnp.float32),
            pltpu.VMEM_SHARED((N_PAD,), jnp.float32),
            pltpu.SemaphoreType.DMA,
            pltpu.SemaphoreType.DMA,
        ],
    )(idx5)

    blk = N_PAD // 8
    g = pl.pallas_call(
        functools.partial(_scale_body, blk=blk),
        grid=(8,),
        in_specs=[
            pl.BlockSpec((NC, 2, blk), lambda i: (0, 0, i)),
            pl.BlockSpec((blk, D_FEAT), lambda i: (i, 0)),
        ],
        out_specs=pl.BlockSpec((blk, D_FEAT), lambda i: (i, 0)),
        out_shape=jax.ShapeDtypeStruct((N_PAD, D_FEAT), jnp.float32),
    )(hist, node_embeddings)

    partials = pl.kernel(
        functools.partial(_agg_body, nsb=nsb),
        out_type=jax.ShapeDtypeStruct((NC, N_PAD, D_FEAT), jnp.float32),
        mesh=mesh,
        scratch_types=[
            pltpu.VMEM((SB, 2, CH), jnp.int32),
            pltpu.VMEM((SB, 2, CH), jnp.int32),
            pltpu.VMEM((CH, D_FEAT), jnp.float32),
            pltpu.VMEM((CH, D_FEAT), jnp.float32),
            pltpu.VMEM_SHARED((N_PAD, D_FEAT), jnp.float32),
            pltpu.SemaphoreType.DMA,
            pltpu.SemaphoreType.DMA,
            pltpu.SemaphoreType.DMA,
            pltpu.SemaphoreType.DMA,
        ],
    )(g, idx5)

    out = pl.pallas_call(
        _combine_body,
        grid=(8,),
        in_specs=[
            pl.BlockSpec((NC, 2, blk), lambda i: (0, 0, i)),
            pl.BlockSpec((NC, blk, D_FEAT), lambda i: (0, i, 0)),
        ],
        out_specs=pl.BlockSpec((blk, D_FEAT), lambda i: (i, 0)),
        out_shape=jax.ShapeDtypeStruct((N_NODES, D_FEAT), jnp.float32),
    )(hist, partials)

    return out
